# Initial kernel scaffold; baseline (speedup 1.0000x reference)
import jax
import jax.numpy as jnp
from jax.experimental import pallas as pl


def kernel(z, W):
    raise NotImplementedError("write your pallas kernel here")


# speedup vs baseline: 1.0943x; 1.0943x over previous
import functools

import jax
import jax.numpy as jnp
from jax import lax
from jax.experimental import pallas as pl
from jax.experimental.pallas import tpu as pltpu
from jax.experimental.pallas import tpu_sc as plsc

N_E = 8192
E_DIM = 32
BETA = 0.25

R_BLK = 512
C_BLK = 4096


def _rownorm(x):
    n = jnp.linalg.norm(x, axis=-1, keepdims=True)
    return x / jnp.maximum(n, 1e-12)


def _copy_body(src_ref, dst_ref):
    dst_ref[...] = src_ref[...]


def _argmin_body(zn_ref, wn_ref, idx_ref, loss_ref, dmin_s, imin_s, carry_s):
    r = pl.program_id(0)
    c = pl.program_id(1)
    nr = pl.num_programs(0)
    nc = pl.num_programs(1)

    z_n = zn_ref[...]
    w_n = wn_ref[...]
    zz = jnp.sum(z_n * z_n, axis=1, keepdims=True)
    ww = jnp.sum(w_n * w_n, axis=1)

    dots = lax.dot_general(
        z_n.astype(jnp.bfloat16), w_n.astype(jnp.bfloat16),
        (((1,), (1,)), ((), ())),
        preferred_element_type=jnp.float32,
    )
    d = (zz + ww[None, :]) - 2.0 * dots

    t_min = jnp.min(d, axis=1, keepdims=True)
    iot = lax.broadcasted_iota(jnp.int32, (R_BLK, C_BLK), 1)
    t_arg = jnp.min(jnp.where(d == t_min, iot, jnp.int32(C_BLK)),
                    axis=1, keepdims=True) + c * C_BLK
    t_min_q = t_min.astype(jnp.bfloat16).astype(jnp.float32)

    @pl.when(c == 0)
    def _():
        dmin_s[...] = t_min
        imin_s[...] = t_arg
        carry_s[...] = t_min_q

    @pl.when(c > 0)
    def _():
        upd = t_min < carry_s[...]
        imin_s[...] = jnp.where(upd, t_arg, imin_s[...])
        dmin_s[...] = jnp.where(upd, t_min, dmin_s[...])
        carry_s[...] = jnp.where(upd, t_min_q, carry_s[...])

    @pl.when(c == nc - 1)
    def _():
        idx_ref[...] = imin_s[...]
        part = jnp.sum(dmin_s[...])
        acc = jnp.where(r == 0, part, loss_ref[...] + part)
        m = acc / jnp.float32(16384 * 32)
        loss_ref[...] = jnp.where(r == nr - 1, BETA * m + m, acc)


def _make_sc_gather(V, D, B):
    info = plsc.get_sparse_core_info()
    n_cores, n_subcores = info.num_cores, info.num_subcores
    nw = n_cores * n_subcores
    b_per_w = B // nw
    mesh = plsc.VectorSubcoreMesh(core_axis_name="c", subcore_axis_name="s")

    @functools.partial(
        pl.kernel, mesh=mesh,
        out_type=jax.ShapeDtypeStruct((B, D), jnp.float32),
        compiler_params=pltpu.CompilerParams(use_tc_tiling_on_sc=False),
        scratch_types=[
            pltpu.VMEM((b_per_w,), jnp.int32),
            pltpu.VMEM((b_per_w, D), jnp.float32),
            pltpu.SemaphoreType.DMA,
        ],
    )
    def gather_k(table_hbm, idx_hbm, out_hbm, idx_v, rows_v, sem):
        wid = lax.axis_index("s") * n_cores + lax.axis_index("c")
        base = wid * b_per_w
        pltpu.sync_copy(idx_hbm.at[pl.ds(base, b_per_w)], idx_v)
        pltpu.async_copy(table_hbm.at[idx_v], rows_v, sem).wait()
        pltpu.sync_copy(rows_v, out_hbm.at[pl.ds(base, b_per_w)])

    return gather_k


def kernel(z, W):
    z = z.astype(jnp.float32)
    W = W.astype(jnp.float32)
    zf = z.reshape(-1, E_DIM)
    B = zf.shape[0]

    zn = _rownorm(zf)
    wn = _rownorm(W)

    idx2, loss2 = pl.pallas_call(
        _argmin_body,
        grid=(B // R_BLK, N_E // C_BLK),
        in_specs=[
            pl.BlockSpec((R_BLK, E_DIM), lambda r, c: (r, 0)),
            pl.BlockSpec((C_BLK, E_DIM), lambda r, c: (c, 0)),
        ],
        out_specs=[
            pl.BlockSpec((R_BLK, 1), lambda r, c: (r, 0)),
            pl.BlockSpec((1, 1), lambda r, c: (0, 0)),
        ],
        out_shape=[
            jax.ShapeDtypeStruct((B, 1), jnp.int32),
            jax.ShapeDtypeStruct((1, 1), jnp.float32),
        ],
        scratch_shapes=[
            pltpu.VMEM((R_BLK, 1), jnp.float32),
            pltpu.VMEM((R_BLK, 1), jnp.int32),
            pltpu.VMEM((R_BLK, 1), jnp.float32),
        ],
    )(zn, wn)

    wn_tab = pl.pallas_call(
        _copy_body,
        grid=(N_E // C_BLK,),
        in_specs=[pl.BlockSpec((C_BLK, E_DIM), lambda i: (i, 0))],
        out_specs=pl.BlockSpec((C_BLK, E_DIM), lambda i: (i, 0)),
        out_shape=jax.ShapeDtypeStruct((N_E, E_DIM), jnp.float32),
    )(wn)

    idx = idx2.reshape(-1)
    zq = _make_sc_gather(N_E, E_DIM, B)(wn_tab, idx)

    return (zq.reshape(z.shape), loss2[0, 0], idx.reshape(z.shape[:-1]))


# speedup vs baseline: 1.2010x; 1.0975x over previous
import functools

import jax
import jax.numpy as jnp
from jax import lax
from jax.experimental import pallas as pl
from jax.experimental.pallas import tpu as pltpu
from jax.experimental.pallas import tpu_sc as plsc

N_E = 8192
E_DIM = 32
BETA = 0.25

R_BLK = 1024
C_BLK = 4096


def _rownorm(x):
    n = jnp.linalg.norm(x, axis=-1, keepdims=True)
    return x / jnp.maximum(n, 1e-12)


def _argmin_body(zn_ref, wn_ref, idx_ref, loss_ref, wt_ref):
    r = pl.program_id(0)
    nr = pl.num_programs(0)

    z_n = zn_ref[...]
    w_n = wn_ref[...]
    zz = jnp.sum(z_n * z_n, axis=1, keepdims=True)

    @pl.when(r == 0)
    def _():
        wt_ref[...] = w_n

    best_d = best_i = carry = None
    iot = lax.broadcasted_iota(jnp.int32, (R_BLK, C_BLK), 1)
    for c in range(N_E // C_BLK):
        w_c = w_n[c * C_BLK:(c + 1) * C_BLK, :]
        ww = jnp.sum(w_c * w_c, axis=1)
        dots = lax.dot_general(
            z_n.astype(jnp.bfloat16), w_c.astype(jnp.bfloat16),
            (((1,), (1,)), ((), ())),
            preferred_element_type=jnp.float32,
        )
        d = (zz + ww[None, :]) - 2.0 * dots
        t_min = jnp.min(d, axis=1, keepdims=True)
        t_arg = jnp.min(jnp.where(d == t_min, iot, jnp.int32(C_BLK)),
                        axis=1, keepdims=True) + c * C_BLK
        t_q = t_min.astype(jnp.bfloat16).astype(jnp.float32)
        if c == 0:
            best_d, best_i, carry = t_min, t_arg, t_q
        else:
            upd = t_min < carry
            best_i = jnp.where(upd, t_arg, best_i)
            best_d = jnp.where(upd, t_min, best_d)
            carry = jnp.where(upd, t_q, carry)

    idx_ref[...] = best_i
    part = jnp.sum(best_d)
    acc = jnp.where(r == 0, part, loss_ref[...] + part)
    m = acc / jnp.float32(16384 * 32)
    loss_ref[...] = jnp.where(r == nr - 1, BETA * m + m, acc)


def _make_sc_gather(V, D, B):
    info = plsc.get_sparse_core_info()
    n_cores, n_subcores = info.num_cores, info.num_subcores
    nw = n_cores * n_subcores
    b_per_w = B // nw
    mesh = plsc.VectorSubcoreMesh(core_axis_name="c", subcore_axis_name="s")

    @functools.partial(
        pl.kernel, mesh=mesh,
        out_type=jax.ShapeDtypeStruct((B, D), jnp.float32),
        compiler_params=pltpu.CompilerParams(use_tc_tiling_on_sc=False),
        scratch_types=[
            pltpu.VMEM((b_per_w,), jnp.int32),
            pltpu.VMEM((b_per_w, D), jnp.float32),
            pltpu.SemaphoreType.DMA,
        ],
    )
    def gather_k(table_hbm, idx_hbm, out_hbm, idx_v, rows_v, sem):
        wid = lax.axis_index("s") * n_cores + lax.axis_index("c")
        base = wid * b_per_w
        pltpu.sync_copy(idx_hbm.at[pl.ds(base, b_per_w)], idx_v)
        pltpu.async_copy(table_hbm.at[idx_v], rows_v, sem).wait()
        pltpu.sync_copy(rows_v, out_hbm.at[pl.ds(base, b_per_w)])

    return gather_k


def kernel(z, W):
    z = z.astype(jnp.float32)
    W = W.astype(jnp.float32)
    zf = z.reshape(-1, E_DIM)
    B = zf.shape[0]

    zn = _rownorm(zf)
    wn = _rownorm(W)

    idx2, loss2, wn_tab = pl.pallas_call(
        _argmin_body,
        grid=(B // R_BLK,),
        in_specs=[
            pl.BlockSpec((R_BLK, E_DIM), lambda r: (r, 0)),
            pl.BlockSpec((N_E, E_DIM), lambda r: (0, 0)),
        ],
        out_specs=[
            pl.BlockSpec((R_BLK, 1), lambda r: (r, 0)),
            pl.BlockSpec((1, 1), lambda r: (0, 0)),
            pl.BlockSpec((N_E, E_DIM), lambda r: (0, 0)),
        ],
        out_shape=[
            jax.ShapeDtypeStruct((B, 1), jnp.int32),
            jax.ShapeDtypeStruct((1, 1), jnp.float32),
            jax.ShapeDtypeStruct((N_E, E_DIM), jnp.float32),
        ],
    )(zn, wn)

    idx = idx2.reshape(-1)
    zq = _make_sc_gather(N_E, E_DIM, B)(wn_tab, idx)

    return (zq.reshape(z.shape), loss2[0, 0], idx.reshape(z.shape[:-1]))
